# baseline (speedup 1.0000x reference)
import functools

import jax
import jax.numpy as jnp
from jax import lax
from jax.experimental import pallas as pl
from jax.experimental.pallas import tpu as pltpu
from jax.experimental.pallas import tpu_sc as plsc

B = 16384
EMB = 32
LANES = 16
PACK = 4

_info = plsc.get_sparse_core_info()
NC = _info.num_cores
NS = _info.num_subcores
NW = NC * NS
BPW = B // NW
CHUNK = 128
HALF = BPW // 2

_mesh = plsc.VectorSubcoreMesh(core_axis_name="c", subcore_axis_name="s")


@functools.partial(
    pl.kernel,
    mesh=_mesh,
    out_type=jax.ShapeDtypeStruct((B,), jnp.float32),
    compiler_params=pltpu.CompilerParams(
        needs_layout_passes=False, use_tc_tiling_on_sc=True),
    scratch_types=[
        pltpu.VMEM((BPW,), jnp.int32),
        pltpu.VMEM((BPW,), jnp.int32),
        pltpu.VMEM((BPW,), jnp.int32),
        pltpu.VMEM((BPW,), jnp.int32),
        pltpu.VMEM((HALF, CHUNK), jnp.float32),
        pltpu.VMEM((HALF, CHUNK), jnp.float32),
        pltpu.VMEM((BPW,), jnp.float32),
        pltpu.SemaphoreType.DMA,
    ],
)
def _mf_sc(u_id_hbm, i_id_hbm, ut_hbm, it_hbm, out_hbm,
           upk_v, ipk_v, uof_v, iof_v, urows_v, irows_v, out_v, sem):
    wid = lax.axis_index("s") * NC + lax.axis_index("c")
    base = wid * BPW

    pltpu.sync_copy(u_id_hbm.at[pl.ds(base, BPW)], uof_v)
    pltpu.sync_copy(i_id_hbm.at[pl.ds(base, BPW)], iof_v)

    for k in range(BPW // LANES):
        sl = pl.ds(k * LANES, LANES)
        uv = uof_v[sl]
        iv = iof_v[sl]
        upk_v[sl] = lax.shift_right_logical(uv, 2)
        ipk_v[sl] = lax.shift_right_logical(iv, 2)
        uof_v[sl] = lax.shift_left(uv & 3, 5)
        iof_v[sl] = lax.shift_left(iv & 3, 5)

    rows0 = jnp.arange(LANES, dtype=jnp.int32)
    for h in range(BPW // HALF):
        copies = []
        for j in range(HALF // CHUNK):
            copies.append(pltpu.async_copy(
                ut_hbm.at[upk_v.at[pl.ds(h * HALF + j * CHUNK, CHUNK)]],
                urows_v.at[pl.ds(j * CHUNK, CHUNK)], sem))
            copies.append(pltpu.async_copy(
                it_hbm.at[ipk_v.at[pl.ds(h * HALF + j * CHUNK, CHUNK)]],
                irows_v.at[pl.ds(j * CHUNK, CHUNK)], sem))
        for c in copies:
            c.wait()

        for k in range(HALF // LANES):
            rows = rows0 + (k * LANES)
            ucols = uof_v[pl.ds(h * HALF + k * LANES, LANES)]
            icols = iof_v[pl.ds(h * HALF + k * LANES, LANES)]
            acc = jnp.zeros((LANES,), jnp.float32)
            for c in range(EMB):
                uv = plsc.load_gather(urows_v, [rows, ucols])
                iv = plsc.load_gather(irows_v, [rows, icols])
                acc = acc + uv * iv
                if c != EMB - 1:
                    ucols = ucols + 1
                    icols = icols + 1
            out_v[pl.ds(h * HALF + k * LANES, LANES)] = acc

    pltpu.sync_copy(out_v, out_hbm.at[pl.ds(base, BPW)])


def kernel(u_id, i_id, user_table, item_table):
    nu, ni = user_table.shape[0], item_table.shape[0]
    ut_p = user_table.reshape(nu // PACK, PACK * EMB)
    it_p = item_table.reshape(ni // PACK, PACK * EMB)
    return _mf_sc(u_id.astype(jnp.int32), i_id.astype(jnp.int32), ut_p, it_p)
